# baseline (speedup 1.0000x reference)
import math

import jax
import jax.numpy as jnp
from jax.experimental import pallas as pl
from jax.experimental.pallas import tpu as pltpu

_TEMP = 0.05
_BN = 512
_LN2 = math.log(2.0)
_SCALE = math.log2(math.e) / _TEMP


def _ce_kernel(x_ref, f_ref, t_ref, out_ref, xn_ref, l_ref, s_ref, tacc_ref):
    j = pl.program_id(0)
    nj = pl.num_programs(0) - 1
    bn = f_ref.shape[0]

    @pl.when(j == 0)
    def _init():
        x = x_ref[...]
        norm2 = jnp.sum(x * x, axis=1, keepdims=True)
        xn = x * (_SCALE * jax.lax.rsqrt(norm2))
        xn_ref[...] = xn.astype(jnp.bfloat16)
        l_ref[1] = jnp.zeros_like(l_ref[1])
        s_ref[...] = jnp.zeros_like(s_ref)
        tacc_ref[...] = jnp.zeros_like(tacc_ref)

    cur = jax.lax.rem(j, 2)
    prev = 1 - cur
    fb = f_ref[...].astype(jnp.bfloat16)
    l_ref[cur] = jax.lax.dot_general(
        xn_ref[...], fb, (((1,), (1,)), ((), ())),
        preferred_element_type=jnp.float32,
    )
    logits = l_ref[prev]
    s_ref[...] += jnp.sum(jnp.exp2(logits), axis=1, keepdims=True)
    cols = (j - 1) * bn + jax.lax.broadcasted_iota(jnp.int32, logits.shape, 1)
    masked = jnp.where(cols == t_ref[...], logits, 0.0)
    tacc_ref[...] += jnp.sum(masked, axis=1, keepdims=True)

    @pl.when(j == nj)
    def _fin():
        per_row = (jnp.log2(s_ref[...] - bn) - tacc_ref[...]) * _LN2
        out_ref[...] = jnp.sum(per_row, keepdims=True) * (1.0 / per_row.shape[0])


def kernel(epoch, inputs, ema_inputs, part_out, score, targets, features,
           part_features):
    m, k = inputs.shape
    n = features.shape[0]
    nj = n // _BN
    out = pl.pallas_call(
        _ce_kernel,
        grid=(nj + 1,),
        in_specs=[
            pl.BlockSpec((m, k), lambda j: (0, 0)),
            pl.BlockSpec((_BN, k), lambda j: (jnp.minimum(j, nj - 1), 0)),
            pl.BlockSpec((m, 1), lambda j: (0, 0)),
        ],
        out_specs=pl.BlockSpec((1, 1), lambda j: (0, 0)),
        out_shape=jax.ShapeDtypeStruct((1, 1), jnp.float32),
        scratch_shapes=[
            pltpu.VMEM((m, k), jnp.bfloat16),
            pltpu.VMEM((2, m, _BN), jnp.float32),
            pltpu.VMEM((m, 1), jnp.float32),
            pltpu.VMEM((m, 1), jnp.float32),
        ],
    )(inputs, features, targets.reshape(m, 1))
    return out[0, 0]


# speedup vs baseline: 1.3108x; 1.3108x over previous
import math

import jax
import jax.numpy as jnp
from jax.experimental import pallas as pl
from jax.experimental.pallas import tpu as pltpu

_TEMP = 0.05
_BN = 512
_LN2 = math.log(2.0)
_SCALE = math.log2(math.e) / _TEMP


def _ce_kernel(x_ref, f_ref, t_ref, out_ref, xn_ref, la_ref, lb_ref, s_ref,
               tacc_ref):
    j = pl.program_id(0)
    nj = pl.num_programs(0)
    bn = la_ref.shape[1]

    @pl.when(j == 0)
    def _init():
        x = x_ref[...]
        norm2 = jnp.sum(x * x, axis=1, keepdims=True)
        xn = x * (_SCALE * jax.lax.rsqrt(norm2))
        xn_ref[...] = xn.astype(jnp.bfloat16)
        lb_ref[...] = jnp.zeros_like(lb_ref)
        s_ref[...] = jnp.zeros_like(s_ref)
        tacc_ref[...] = jnp.zeros_like(tacc_ref)

    def _reduce(l_ref, blk):
        logits = l_ref[...]
        s_ref[...] += jnp.sum(jnp.exp2(logits), axis=1, keepdims=True)
        cols = blk * bn + jax.lax.broadcasted_iota(jnp.int32, logits.shape, 1)
        masked = jnp.where(cols == t_ref[...], logits, 0.0)
        tacc_ref[...] += jnp.sum(masked, axis=1, keepdims=True)

    xn = xn_ref[...]
    fb = f_ref[...].astype(jnp.bfloat16)
    la_ref[...] = jax.lax.dot_general(
        xn, fb[:bn], (((1,), (1,)), ((), ())),
        preferred_element_type=jnp.float32,
    )
    _reduce(lb_ref, 2 * j - 1)
    lb_ref[...] = jax.lax.dot_general(
        xn, fb[bn:], (((1,), (1,)), ((), ())),
        preferred_element_type=jnp.float32,
    )
    _reduce(la_ref, 2 * j)

    @pl.when(j == nj - 1)
    def _fin():
        _reduce(lb_ref, 2 * j + 1)
        per_row = (jnp.log2(s_ref[...] - bn) - tacc_ref[...]) * _LN2
        out_ref[...] = jnp.sum(per_row, keepdims=True) * (1.0 / per_row.shape[0])


def kernel(epoch, inputs, ema_inputs, part_out, score, targets, features,
           part_features):
    m, k = inputs.shape
    n = features.shape[0]
    nj = n // (2 * _BN)
    out = pl.pallas_call(
        _ce_kernel,
        grid=(nj,),
        in_specs=[
            pl.BlockSpec((m, k), lambda j: (0, 0)),
            pl.BlockSpec((2 * _BN, k), lambda j: (j, 0)),
            pl.BlockSpec((m, 1), lambda j: (0, 0)),
        ],
        out_specs=pl.BlockSpec((1, 1), lambda j: (0, 0)),
        out_shape=jax.ShapeDtypeStruct((1, 1), jnp.float32),
        scratch_shapes=[
            pltpu.VMEM((m, k), jnp.bfloat16),
            pltpu.VMEM((m, _BN), jnp.float32),
            pltpu.VMEM((m, _BN), jnp.float32),
            pltpu.VMEM((m, 1), jnp.float32),
            pltpu.VMEM((m, 1), jnp.float32),
        ],
    )(inputs, features, targets.reshape(m, 1))
    return out[0, 0]


# speedup vs baseline: 1.3370x; 1.0200x over previous
import math

import jax
import jax.numpy as jnp
from jax.experimental import pallas as pl
from jax.experimental.pallas import tpu as pltpu

_TEMP = 0.05
_BN = 1024
_LN2 = math.log(2.0)
_SCALE = math.log2(math.e) / _TEMP


def _ce_kernel(x_ref, f_ref, t_ref, out_ref, xn_ref, s_ref, tacc_ref):
    j = pl.program_id(0)
    nj = pl.num_programs(0)
    bn = f_ref.shape[0]

    @pl.when(j == 0)
    def _init():
        x = x_ref[...]
        norm2 = jnp.sum(x * x, axis=1, keepdims=True)
        xn_ref[...] = x * (_SCALE * jax.lax.rsqrt(norm2))
        s_ref[...] = jnp.zeros_like(s_ref)
        tacc_ref[...] = jnp.zeros_like(tacc_ref)

    logits = jax.lax.dot_general(
        xn_ref[...], f_ref[...], (((1,), (1,)), ((), ())),
        preferred_element_type=jnp.float32,
    )
    s_ref[...] += jnp.sum(jnp.exp2(logits), axis=1, keepdims=True)
    cols = j * bn + jax.lax.broadcasted_iota(jnp.int32, logits.shape, 1)
    masked = jnp.where(cols == t_ref[...], logits, 0.0)
    tacc_ref[...] += jnp.sum(masked, axis=1, keepdims=True)

    @pl.when(j == nj - 1)
    def _fin():
        per_row = (jnp.log2(s_ref[...]) - tacc_ref[...]) * _LN2
        out_ref[...] = jnp.sum(per_row, keepdims=True) * (1.0 / per_row.shape[0])


def kernel(epoch, inputs, ema_inputs, part_out, score, targets, features,
           part_features):
    m, k = inputs.shape
    n = features.shape[0]
    out = pl.pallas_call(
        _ce_kernel,
        grid=(n // _BN,),
        in_specs=[
            pl.BlockSpec((m, k), lambda j: (0, 0)),
            pl.BlockSpec((_BN, k), lambda j: (j, 0)),
            pl.BlockSpec((m, 1), lambda j: (0, 0)),
        ],
        out_specs=pl.BlockSpec((1, 1), lambda j: (0, 0)),
        out_shape=jax.ShapeDtypeStruct((1, 1), jnp.float32),
        scratch_shapes=[
            pltpu.VMEM((m, k), jnp.float32),
            pltpu.VMEM((m, 1), jnp.float32),
            pltpu.VMEM((m, 1), jnp.float32),
        ],
    )(inputs, features, targets.reshape(m, 1))
    return out[0, 0]
